# baseline (speedup 1.0000x reference)
import numpy as np
import jax
import jax.numpy as jnp
from jax.experimental import pallas as pl
from jax.experimental.pallas import tpu as pltpu

IMG_SIZE = 512
STRIDES = [8, 16, 32]
ANCHOR_SIZES = [
    [(10.0, 13.0), (16.0, 30.0), (33.0, 23.0)],
    [(30.0, 61.0), (62.0, 45.0), (59.0, 119.0)],
    [(116.0, 90.0), (156.0, 198.0), (373.0, 326.0)],
]
NUM_CLASSES = 80
B = 8
M = 32

_INTERPRET = False

NB = 768
ROWS = NB // 128
P = 4
L0 = 3 * 64 * 64
L1 = 3 * 32 * 32
L2 = 3 * 16 * 16
N = L0 + L1 + L2
NB0 = L0 // NB
NB1 = L1 // NB
NB2 = L2 // NB
NB_TOT = NB0 + NB1 + NB2
NBP = (NB_TOT + P - 1) // P
NB_PAD = NBP * P


def _make_anchor_table() -> np.ndarray:
    comps = [[] for _ in range(8)]
    for stride, sizes in zip(STRIDES, ANCHOR_SIZES):
        g = IMG_SIZE // stride
        ys, xs = np.meshgrid(np.arange(g, dtype=np.float32),
                             np.arange(g, dtype=np.float32), indexing='ij')
        cx = (xs + 0.5) * stride
        cy = (ys + 0.5) * stride
        for (aw, ah) in sizes:
            x1 = (cx - aw / 2).reshape(-1)
            y1 = (cy - ah / 2).reshape(-1)
            x2 = (cx + aw / 2).reshape(-1)
            y2 = (cy + ah / 2).reshape(-1)
            vals = [x1, y1, x2, y2, (x1 + x2) / 2, (y1 + y2) / 2,
                    np.full_like(x1, aw), np.full_like(x1, ah)]
            for i in range(8):
                comps[i].append(vals[i])
    flat = np.stack([np.concatenate(c) for c in comps], axis=0)
    blocked = flat.reshape(8, NB_TOT, ROWS, 128)
    pad_row = np.zeros((8, NB_TOT, 8 - ROWS, 128), np.float32)
    pad_row[6:8] = 1.0
    out = np.concatenate([blocked, pad_row], axis=2)
    pad_blk = np.zeros((8, NB_PAD - NB_TOT, 8, 128), np.float32)
    pad_blk[6:8] = 1.0
    out = np.concatenate([out, pad_blk], axis=1)
    return np.ascontiguousarray(out.transpose(1, 0, 2, 3)).astype(np.float32)


_ANCHORS = _make_anchor_table()


def _softplus(x):
    return jnp.maximum(x, 0.0) + jnp.log(1.0 + jnp.exp(-jnp.abs(x)))


def _loss_body(tgt_ref, anch_ref, reg_ref, cls0_ref, cls1_ref, cls2_ref,
               npos_ref, obj_ref, clss_ref, regs_ref,
               npa_ref, oba_ref, rga_ref, cla_ref):
    b = pl.program_id(0)
    nbp = pl.program_id(1)

    @pl.when(jnp.logical_and(nbp == 0, b == 0))
    def _init():
        npa_ref[...] = jnp.zeros_like(npa_ref)
        oba_ref[...] = jnp.zeros_like(oba_ref)
        rga_ref[...] = jnp.zeros_like(rga_ref)
        cla_ref[...] = jnp.zeros_like(cla_ref)

    in_l0 = nbp < NB0 // P
    in_l1 = nbp == NB0 // P
    wobj = (nbp < NBP - 1).astype(jnp.float32)

    rowmask = (jax.lax.broadcasted_iota(jnp.int32, (8, 128), 0)
               < ROWS).astype(jnp.float32)
    citer = jax.lax.broadcasted_iota(jnp.int32, (128, NUM_CLASSES), 1)

    for p in range(P):
        nb = nbp * P + p

        ax1 = anch_ref[nb, 0]
        ay1 = anch_ref[nb, 1]
        ax2 = anch_ref[nb, 2]
        ay2 = anch_ref[nb, 3]
        acx = anch_ref[nb, 4]
        acy = anch_ref[nb, 5]
        aw = anch_ref[nb, 6]
        ah = anch_ref[nb, 7]
        area_a = (ax2 - ax1) * (ay2 - ay1)

        best_int = jnp.full((8, 128), -1.0, dtype=jnp.float32)
        best_den = jnp.ones((8, 128), dtype=jnp.float32)
        mgcx = jnp.zeros((8, 128), dtype=jnp.float32)
        mgcy = jnp.zeros((8, 128), dtype=jnp.float32)
        mgw = jnp.full((8, 128), 1e-3, dtype=jnp.float32)
        mgh = jnp.full((8, 128), 1e-3, dtype=jnp.float32)
        mcls = jnp.zeros((8, 128), dtype=jnp.float32)

        for m in range(M):
            gx1 = jnp.full((8, 128), tgt_ref[0, 0, 0 * M + m])
            gy1 = jnp.full((8, 128), tgt_ref[0, 0, 1 * M + m])
            gx2 = jnp.full((8, 128), tgt_ref[0, 0, 2 * M + m])
            gy2 = jnp.full((8, 128), tgt_ref[0, 0, 3 * M + m])
            area_b = jnp.full((8, 128), tgt_ref[0, 0, 4 * M + m])
            gcl = jnp.full((8, 128), tgt_ref[0, 0, 5 * M + m])

            iw = jnp.clip(jnp.minimum(ax2, gx2) - jnp.maximum(ax1, gx1), 0.0)
            ih = jnp.clip(jnp.minimum(ay2, gy2) - jnp.maximum(ay1, gy1), 0.0)
            inter = iw * ih
            denom = area_a + area_b - inter + 1e-9

            better = inter * best_den > best_int * denom
            best_int = jnp.where(better, inter, best_int)
            best_den = jnp.where(better, denom, best_den)
            mgcx = jnp.where(better, (gx1 + gx2) * 0.5, mgcx)
            mgcy = jnp.where(better, (gy1 + gy2) * 0.5, mgcy)
            mgw = jnp.where(better, jnp.maximum(gx2 - gx1, 1e-3), mgw)
            mgh = jnp.where(better, jnp.maximum(gy2 - gy1, 1e-3), mgh)
            mcls = jnp.where(better, gcl, mcls)

        posf = (best_int > 0.5 * best_den).astype(jnp.float32)

        regv = reg_ref[0, p]

        rt0 = (mgcx - acx) / aw
        rt1 = (mgcy - acy) / ah
        rt2 = jnp.log(mgw / aw)
        rt3 = jnp.log(mgh / ah)

        def sl1(d):
            ad = jnp.abs(d)
            return jnp.where(ad < 1.0, 0.5 * d * d, ad - 0.5)

        reg_row = (sl1(regv[0] - rt0) + sl1(regv[1] - rt1)
                   + sl1(regv[2] - rt2) + sl1(regv[3] - rt3))
        obj_pred = regv[4]
        obj_bce = (_softplus(obj_pred) - obj_pred * posf) * rowmask

        npa_ref[...] += posf
        rga_ref[...] += reg_row * posf
        if p == 0:
            oba_ref[...] += obj_bce
        else:
            oba_ref[...] += obj_bce * wobj

        has_pos = jnp.max(posf) > 0.5
        comb_t = jnp.transpose(mcls[0:ROWS]
                               + 128.0 * posf[0:ROWS])

        def _cls_phase(ref, base):
            acc = jnp.zeros((128, NUM_CLASSES), jnp.float32)
            for r in range(ROWS):
                cc = comb_t[:, r:r + 1]
                pf = (cc >= 128.0).astype(jnp.float32)
                cid_i = (cc - 128.0 * pf + 0.5).astype(jnp.int32)
                x = ref[0, base + r * 128: base + (r + 1) * 128, :]
                t = _softplus(x) - jnp.where(citer == cid_i, x, 0.0)
                acc = acc + t * pf
            cla_ref[...] += acc

        @pl.when(jnp.logical_and(has_pos, in_l0))
        def _c0(p=p):
            _cls_phase(cls0_ref, p * NB)

        @pl.when(jnp.logical_and(has_pos, in_l1))
        def _c1(p=p):
            _cls_phase(cls1_ref, p * NB)

        if p == 0:
            @pl.when(jnp.logical_and(has_pos, nbp == NBP - 1))
            def _c2():
                _cls_phase(cls2_ref, 0)

    @pl.when(jnp.logical_and(b == B - 1, nbp == NBP - 1))
    def _fin():
        npos_ref[...] = jnp.sum(npa_ref[...]).reshape(1, 1)
        obj_ref[...] = jnp.sum(oba_ref[...]).reshape(1, 1)
        regs_ref[...] = jnp.sum(rga_ref[...]).reshape(1, 1)
        clss_ref[...] = jnp.sum(cla_ref[...]).reshape(1, 1)


@jax.jit
def _loss_pallas(tgt_s, reg_pad, cls0, cls1, cls2):
    anchors = jnp.asarray(_ANCHORS)
    grid = (B, NBP)

    out = pl.pallas_call(
        _loss_body,
        grid=grid,
        in_specs=[
            pl.BlockSpec((1, 1, 6 * M), lambda b, nbp: (b, 0, 0),
                         memory_space=pltpu.SMEM),
            pl.BlockSpec((NB_PAD, 8, 8, 128), lambda b, nbp: (0, 0, 0, 0)),
            pl.BlockSpec((1, P, 5, 8, 128), lambda b, nbp: (b, nbp, 0, 0, 0)),
            pl.BlockSpec((1, P * NB, NUM_CLASSES),
                         lambda b, nbp: (b, jnp.minimum(nbp, NB0 // P - 1), 0)),
            pl.BlockSpec((1, L1, NUM_CLASSES), lambda b, nbp: (b, 0, 0)),
            pl.BlockSpec((1, L2, NUM_CLASSES), lambda b, nbp: (b, 0, 0)),
        ],
        out_specs=[pl.BlockSpec((1, 1), lambda b, nbp: (0, 0))] * 4,
        out_shape=[jax.ShapeDtypeStruct((1, 1), jnp.float32)] * 4,
        scratch_shapes=[
            pltpu.VMEM((8, 128), jnp.float32),
            pltpu.VMEM((8, 128), jnp.float32),
            pltpu.VMEM((8, 128), jnp.float32),
            pltpu.VMEM((128, NUM_CLASSES), jnp.float32),
        ],
        compiler_params=pltpu.CompilerParams(
            dimension_semantics=("arbitrary", "arbitrary")),
        interpret=_INTERPRET,
    )(tgt_s, anchors, reg_pad, cls0, cls1, cls2)
    return out


def kernel(imgs, reg_l0, reg_l1, reg_l2, cls_l0, cls_l1, cls_l2, targets):
    del imgs

    def regt(x, nblk):
        r = jnp.transpose(x.reshape(B, nblk, ROWS, 128, 5), (0, 1, 4, 2, 3))
        return jnp.concatenate(
            [r, jnp.zeros((B, nblk, 5, 8 - ROWS, 128), jnp.float32)], axis=3)

    reg_pad = jnp.concatenate(
        [regt(reg_l0, NB0), regt(reg_l1, NB1), regt(reg_l2, NB2),
         jnp.zeros((B, NB_PAD - NB_TOT, 5, 8, 128), jnp.float32)], axis=1)

    cls0 = cls_l0.reshape(B, L0, NUM_CLASSES)
    cls1 = cls_l1.reshape(B, L1, NUM_CLASSES)
    cls2 = cls_l2.reshape(B, L2, NUM_CLASSES)

    gx1 = targets[..., 0]
    gy1 = targets[..., 1]
    gx2 = targets[..., 2]
    gy2 = targets[..., 3]
    gcl = targets[..., 4]
    valid = jnp.logical_and(gx2 > gx1, gy2 > gy1)
    area_b = jnp.clip(gx2 - gx1, 0.0) * jnp.clip(gy2 - gy1, 0.0)
    area_b = jnp.where(valid, area_b, 1e30)
    tgt_s = jnp.stack(
        [gx1, gy1, gx2, gy2, area_b, gcl], axis=1).reshape(B, 1, 6 * M)

    npos_s, obj_s, cls_s, reg_s = _loss_pallas(
        tgt_s, reg_pad, cls0, cls1, cls2)

    npos = jnp.maximum(npos_s[0, 0], 1.0)
    loss_obj = obj_s[0, 0] / (B * N)
    loss_cls = cls_s[0, 0] / npos
    loss_reg = reg_s[0, 0] / npos
    losses = loss_reg + loss_obj + loss_cls
    return (losses, loss_reg, loss_obj, loss_cls)


# speedup vs baseline: 1.1322x; 1.1322x over previous
import numpy as np
import jax
import jax.numpy as jnp
from jax.experimental import pallas as pl
from jax.experimental.pallas import tpu as pltpu

IMG_SIZE = 512
STRIDES = [8, 16, 32]
ANCHOR_SIZES = [
    [(10.0, 13.0), (16.0, 30.0), (33.0, 23.0)],
    [(30.0, 61.0), (62.0, 45.0), (59.0, 119.0)],
    [(116.0, 90.0), (156.0, 198.0), (373.0, 326.0)],
]
NUM_CLASSES = 80
B = 8
M = 32

_INTERPRET = False

NB = 768
ROWS = NB // 128
P = 8
L0 = 3 * 64 * 64
L1 = 3 * 32 * 32
L2 = 3 * 16 * 16
N = L0 + L1 + L2
NB0 = L0 // NB
NB1 = L1 // NB
NB2 = L2 // NB
NB_TOT = NB0 + NB1 + NB2
NBP = (NB_TOT + P - 1) // P
NB_PAD = NBP * P


def _make_anchor_table() -> np.ndarray:
    comps = [[] for _ in range(8)]
    for stride, sizes in zip(STRIDES, ANCHOR_SIZES):
        g = IMG_SIZE // stride
        ys, xs = np.meshgrid(np.arange(g, dtype=np.float32),
                             np.arange(g, dtype=np.float32), indexing='ij')
        cx = (xs + 0.5) * stride
        cy = (ys + 0.5) * stride
        for (aw, ah) in sizes:
            x1 = (cx - aw / 2).reshape(-1)
            y1 = (cy - ah / 2).reshape(-1)
            x2 = (cx + aw / 2).reshape(-1)
            y2 = (cy + ah / 2).reshape(-1)
            vals = [x1, y1, x2, y2, (x1 + x2) / 2, (y1 + y2) / 2,
                    np.full_like(x1, aw), np.full_like(x1, ah)]
            for i in range(8):
                comps[i].append(vals[i])
    flat = np.stack([np.concatenate(c) for c in comps], axis=0)
    blocked = flat.reshape(8, NB_TOT, ROWS, 128)
    pad_row = np.zeros((8, NB_TOT, 8 - ROWS, 128), np.float32)
    pad_row[6:8] = 1.0
    out = np.concatenate([blocked, pad_row], axis=2)
    pad_blk = np.zeros((8, NB_PAD - NB_TOT, 8, 128), np.float32)
    pad_blk[6:8] = 1.0
    out = np.concatenate([out, pad_blk], axis=1)
    return np.ascontiguousarray(out.transpose(1, 0, 2, 3)).astype(np.float32)


_ANCHORS = _make_anchor_table()


def _softplus(x):
    return jnp.maximum(x, 0.0) + jnp.log(1.0 + jnp.exp(-jnp.abs(x)))


def _loss_body(tgt_ref, anch_ref, reg_ref, cls0_ref, cls1_ref, cls2_ref,
               npos_ref, obj_ref, clss_ref, regs_ref,
               npa_ref, oba_ref, rga_ref, cla_ref):
    b = pl.program_id(0)
    nbp = pl.program_id(1)

    @pl.when(jnp.logical_and(nbp == 0, b == 0))
    def _init():
        npa_ref[...] = jnp.zeros_like(npa_ref)
        oba_ref[...] = jnp.zeros_like(oba_ref)
        rga_ref[...] = jnp.zeros_like(rga_ref)
        cla_ref[...] = jnp.zeros_like(cla_ref)

    in_l0 = nbp < NB0 // P
    in_l1 = nbp == NB0 // P
    wobj = (nbp < NBP - 1).astype(jnp.float32)

    rowmask = (jax.lax.broadcasted_iota(jnp.int32, (8, 128), 0)
               < ROWS).astype(jnp.float32)
    citer = jax.lax.broadcasted_iota(jnp.int32, (128, NUM_CLASSES), 1)

    for p in range(P):
        nb = nbp * P + p

        ax1 = anch_ref[nb, 0]
        ay1 = anch_ref[nb, 1]
        ax2 = anch_ref[nb, 2]
        ay2 = anch_ref[nb, 3]
        acx = anch_ref[nb, 4]
        acy = anch_ref[nb, 5]
        aw = anch_ref[nb, 6]
        ah = anch_ref[nb, 7]
        area_a = (ax2 - ax1) * (ay2 - ay1)

        best_iou = jnp.full((8, 128), -1.0, dtype=jnp.float32)
        mgcx = jnp.zeros((8, 128), dtype=jnp.float32)
        mgcy = jnp.zeros((8, 128), dtype=jnp.float32)
        mgw = jnp.full((8, 128), 1e-3, dtype=jnp.float32)
        mgh = jnp.full((8, 128), 1e-3, dtype=jnp.float32)
        mcls = jnp.zeros((8, 128), dtype=jnp.float32)

        for m in range(M):
            gx1 = jnp.full((8, 128), tgt_ref[0, 0, 0 * M + m])
            gy1 = jnp.full((8, 128), tgt_ref[0, 0, 1 * M + m])
            gx2 = jnp.full((8, 128), tgt_ref[0, 0, 2 * M + m])
            gy2 = jnp.full((8, 128), tgt_ref[0, 0, 3 * M + m])
            area_b = jnp.full((8, 128), tgt_ref[0, 0, 4 * M + m])
            gcl = jnp.full((8, 128), tgt_ref[0, 0, 5 * M + m])

            iw = jnp.clip(jnp.minimum(ax2, gx2) - jnp.maximum(ax1, gx1), 0.0)
            ih = jnp.clip(jnp.minimum(ay2, gy2) - jnp.maximum(ay1, gy1), 0.0)
            inter = iw * ih
            iou = inter / (area_a + area_b - inter + 1e-9)

            better = iou > best_iou
            best_iou = jnp.where(better, iou, best_iou)
            mgcx = jnp.where(better, (gx1 + gx2) * 0.5, mgcx)
            mgcy = jnp.where(better, (gy1 + gy2) * 0.5, mgcy)
            mgw = jnp.where(better, jnp.maximum(gx2 - gx1, 1e-3), mgw)
            mgh = jnp.where(better, jnp.maximum(gy2 - gy1, 1e-3), mgh)
            mcls = jnp.where(better, gcl, mcls)

        posf = (best_iou > 0.5).astype(jnp.float32)

        regv = reg_ref[0, p]

        rt0 = (mgcx - acx) / aw
        rt1 = (mgcy - acy) / ah
        rt2 = jnp.log(mgw / aw)
        rt3 = jnp.log(mgh / ah)

        def sl1(d):
            ad = jnp.abs(d)
            return jnp.where(ad < 1.0, 0.5 * d * d, ad - 0.5)

        reg_row = (sl1(regv[0] - rt0) + sl1(regv[1] - rt1)
                   + sl1(regv[2] - rt2) + sl1(regv[3] - rt3))
        obj_pred = regv[4]
        obj_bce = (_softplus(obj_pred) - obj_pred * posf) * rowmask

        npa_ref[...] += posf
        rga_ref[...] += reg_row * posf
        if p <= 4:
            oba_ref[...] += obj_bce
        else:
            oba_ref[...] += obj_bce * wobj

        has_pos = jnp.max(best_iou) > 0.5
        comb_t = jnp.transpose(mcls[0:ROWS]
                               + 128.0 * posf[0:ROWS])

        def _cls_phase(ref, base):
            acc = jnp.zeros((128, NUM_CLASSES), jnp.float32)
            for r in range(ROWS):
                cc = comb_t[:, r:r + 1]
                pf = (cc >= 128.0).astype(jnp.float32)
                cid_i = (cc - 128.0 * pf + 0.5).astype(jnp.int32)
                x = ref[0, base + r * 128: base + (r + 1) * 128, :]
                t = _softplus(x) - jnp.where(citer == cid_i, x, 0.0)
                acc = acc + t * pf
            cla_ref[...] += acc

        @pl.when(jnp.logical_and(has_pos, in_l0))
        def _c0(p=p):
            _cls_phase(cls0_ref, p * NB)

        if p < 4:
            @pl.when(jnp.logical_and(has_pos, in_l1))
            def _c1(p=p):
                _cls_phase(cls1_ref, p * NB)

        if p == 4:
            @pl.when(jnp.logical_and(has_pos, nbp == NBP - 1))
            def _c2():
                _cls_phase(cls2_ref, 0)

    @pl.when(jnp.logical_and(b == B - 1, nbp == NBP - 1))
    def _fin():
        npos_ref[...] = jnp.sum(npa_ref[...]).reshape(1, 1)
        obj_ref[...] = jnp.sum(oba_ref[...]).reshape(1, 1)
        regs_ref[...] = jnp.sum(rga_ref[...]).reshape(1, 1)
        clss_ref[...] = jnp.sum(cla_ref[...]).reshape(1, 1)


@jax.jit
def _loss_pallas(tgt_s, reg_pad, cls0, cls1, cls2):
    anchors = jnp.asarray(_ANCHORS)
    grid = (B, NBP)

    out = pl.pallas_call(
        _loss_body,
        grid=grid,
        in_specs=[
            pl.BlockSpec((1, 1, 6 * M), lambda b, nbp: (b, 0, 0),
                         memory_space=pltpu.SMEM),
            pl.BlockSpec((NB_PAD, 8, 8, 128), lambda b, nbp: (0, 0, 0, 0)),
            pl.BlockSpec((1, P, 5, 8, 128), lambda b, nbp: (b, nbp, 0, 0, 0)),
            pl.BlockSpec((1, P * NB, NUM_CLASSES),
                         lambda b, nbp: (b, jnp.minimum(nbp, NB0 // P - 1), 0)),
            pl.BlockSpec((1, L1, NUM_CLASSES), lambda b, nbp: (b, 0, 0)),
            pl.BlockSpec((1, L2, NUM_CLASSES), lambda b, nbp: (b, 0, 0)),
        ],
        out_specs=[pl.BlockSpec((1, 1), lambda b, nbp: (0, 0))] * 4,
        out_shape=[jax.ShapeDtypeStruct((1, 1), jnp.float32)] * 4,
        scratch_shapes=[
            pltpu.VMEM((8, 128), jnp.float32),
            pltpu.VMEM((8, 128), jnp.float32),
            pltpu.VMEM((8, 128), jnp.float32),
            pltpu.VMEM((128, NUM_CLASSES), jnp.float32),
        ],
        compiler_params=pltpu.CompilerParams(
            dimension_semantics=("arbitrary", "arbitrary")),
        interpret=_INTERPRET,
    )(tgt_s, anchors, reg_pad, cls0, cls1, cls2)
    return out


def kernel(imgs, reg_l0, reg_l1, reg_l2, cls_l0, cls_l1, cls_l2, targets):
    del imgs

    def regt(x, nblk):
        r = jnp.transpose(x.reshape(B, nblk, ROWS, 128, 5), (0, 1, 4, 2, 3))
        return jnp.concatenate(
            [r, jnp.zeros((B, nblk, 5, 8 - ROWS, 128), jnp.float32)], axis=3)

    reg_pad = jnp.concatenate(
        [regt(reg_l0, NB0), regt(reg_l1, NB1), regt(reg_l2, NB2),
         jnp.zeros((B, NB_PAD - NB_TOT, 5, 8, 128), jnp.float32)], axis=1)

    cls0 = cls_l0.reshape(B, L0, NUM_CLASSES)
    cls1 = cls_l1.reshape(B, L1, NUM_CLASSES)
    cls2 = cls_l2.reshape(B, L2, NUM_CLASSES)

    gx1 = targets[..., 0]
    gy1 = targets[..., 1]
    gx2 = targets[..., 2]
    gy2 = targets[..., 3]
    gcl = targets[..., 4]
    valid = jnp.logical_and(gx2 > gx1, gy2 > gy1)
    area_b = jnp.clip(gx2 - gx1, 0.0) * jnp.clip(gy2 - gy1, 0.0)
    area_b = jnp.where(valid, area_b, 1e30)
    tgt_s = jnp.stack(
        [gx1, gy1, gx2, gy2, area_b, gcl], axis=1).reshape(B, 1, 6 * M)

    npos_s, obj_s, cls_s, reg_s = _loss_pallas(
        tgt_s, reg_pad, cls0, cls1, cls2)

    npos = jnp.maximum(npos_s[0, 0], 1.0)
    loss_obj = obj_s[0, 0] / (B * N)
    loss_cls = cls_s[0, 0] / npos
    loss_reg = reg_s[0, 0] / npos
    losses = loss_reg + loss_obj + loss_cls
    return (losses, loss_reg, loss_obj, loss_cls)


# speedup vs baseline: 1.2183x; 1.0760x over previous
import numpy as np
import jax
import jax.numpy as jnp
from jax.experimental import pallas as pl
from jax.experimental.pallas import tpu as pltpu

IMG_SIZE = 512
STRIDES = [8, 16, 32]
ANCHOR_SIZES = [
    [(10.0, 13.0), (16.0, 30.0), (33.0, 23.0)],
    [(30.0, 61.0), (62.0, 45.0), (59.0, 119.0)],
    [(116.0, 90.0), (156.0, 198.0), (373.0, 326.0)],
]
NUM_CLASSES = 80
B = 8
M = 32

_INTERPRET = False

NB = 768
ROWS = NB // 128
L0 = 3 * 64 * 64
L1 = 3 * 32 * 32
L2 = 3 * 16 * 16
N = L0 + L1 + L2
NB0 = L0 // NB
NB1 = L1 // NB
NB2 = L2 // NB
NB_TOT = NB0 + NB1 + NB2


def _make_anchor_table() -> np.ndarray:
    comps = [[] for _ in range(8)]
    for stride, sizes in zip(STRIDES, ANCHOR_SIZES):
        g = IMG_SIZE // stride
        ys, xs = np.meshgrid(np.arange(g, dtype=np.float32),
                             np.arange(g, dtype=np.float32), indexing='ij')
        cx = (xs + 0.5) * stride
        cy = (ys + 0.5) * stride
        for (aw, ah) in sizes:
            x1 = (cx - aw / 2).reshape(-1)
            y1 = (cy - ah / 2).reshape(-1)
            x2 = (cx + aw / 2).reshape(-1)
            y2 = (cy + ah / 2).reshape(-1)
            vals = [x1, y1, x2, y2, (x1 + x2) / 2, (y1 + y2) / 2,
                    np.full_like(x1, aw), np.full_like(x1, ah)]
            for i in range(8):
                comps[i].append(vals[i])
    flat = np.stack([np.concatenate(c) for c in comps], axis=0)
    blocked = flat.reshape(8, NB_TOT, ROWS, 128)
    pad_row = np.zeros((8, NB_TOT, 8 - ROWS, 128), np.float32)
    pad_row[6:8] = 1.0
    out = np.concatenate([blocked, pad_row], axis=2)
    return np.ascontiguousarray(out.transpose(1, 0, 2, 3)).astype(np.float32)


_ANCHORS = _make_anchor_table()


def _softplus(x):
    return jnp.maximum(x, 0.0) + jnp.log(1.0 + jnp.exp(-jnp.abs(x)))


def _loss_body(tgt_ref, anch_ref, reg_ref, cls0_ref, cls1_ref, cls2_ref,
               npos_ref, obj_ref, clss_ref, regs_ref,
               npa_ref, oba_ref, rga_ref, cla_ref):
    b = pl.program_id(0)

    @pl.when(b == 0)
    def _init():
        npa_ref[...] = jnp.zeros_like(npa_ref)
        oba_ref[...] = jnp.zeros_like(oba_ref)
        rga_ref[...] = jnp.zeros_like(rga_ref)
        cla_ref[...] = jnp.zeros_like(cla_ref)

    rowmask = (jax.lax.broadcasted_iota(jnp.int32, (8, 128), 0)
               < ROWS).astype(jnp.float32)
    citer = jax.lax.broadcasted_iota(jnp.int32, (128, NUM_CLASSES), 1)

    for nb in range(NB_TOT):

        ax1 = anch_ref[nb, 0]
        ay1 = anch_ref[nb, 1]
        ax2 = anch_ref[nb, 2]
        ay2 = anch_ref[nb, 3]
        acx = anch_ref[nb, 4]
        acy = anch_ref[nb, 5]
        aw = anch_ref[nb, 6]
        ah = anch_ref[nb, 7]
        area_a = (ax2 - ax1) * (ay2 - ay1)

        best_iou = jnp.full((8, 128), -1.0, dtype=jnp.float32)
        mgcx = jnp.zeros((8, 128), dtype=jnp.float32)
        mgcy = jnp.zeros((8, 128), dtype=jnp.float32)
        mgw = jnp.full((8, 128), 1e-3, dtype=jnp.float32)
        mgh = jnp.full((8, 128), 1e-3, dtype=jnp.float32)
        mcls = jnp.zeros((8, 128), dtype=jnp.float32)

        for m in range(M):
            gx1 = jnp.full((8, 128), tgt_ref[0, 0, 0 * M + m])
            gy1 = jnp.full((8, 128), tgt_ref[0, 0, 1 * M + m])
            gx2 = jnp.full((8, 128), tgt_ref[0, 0, 2 * M + m])
            gy2 = jnp.full((8, 128), tgt_ref[0, 0, 3 * M + m])
            area_b = jnp.full((8, 128), tgt_ref[0, 0, 4 * M + m])
            gcl = jnp.full((8, 128), tgt_ref[0, 0, 5 * M + m])

            iw = jnp.clip(jnp.minimum(ax2, gx2) - jnp.maximum(ax1, gx1), 0.0)
            ih = jnp.clip(jnp.minimum(ay2, gy2) - jnp.maximum(ay1, gy1), 0.0)
            inter = iw * ih
            iou = inter / (area_a + area_b - inter + 1e-9)

            better = iou > best_iou
            best_iou = jnp.where(better, iou, best_iou)
            mgcx = jnp.where(better, (gx1 + gx2) * 0.5, mgcx)
            mgcy = jnp.where(better, (gy1 + gy2) * 0.5, mgcy)
            mgw = jnp.where(better, jnp.maximum(gx2 - gx1, 1e-3), mgw)
            mgh = jnp.where(better, jnp.maximum(gy2 - gy1, 1e-3), mgh)
            mcls = jnp.where(better, gcl, mcls)

        posf = (best_iou > 0.5).astype(jnp.float32)

        regv = reg_ref[0, nb]

        rt0 = (mgcx - acx) / aw
        rt1 = (mgcy - acy) / ah
        rt2 = jnp.log(mgw / aw)
        rt3 = jnp.log(mgh / ah)

        def sl1(d):
            ad = jnp.abs(d)
            return jnp.where(ad < 1.0, 0.5 * d * d, ad - 0.5)

        reg_row = (sl1(regv[0] - rt0) + sl1(regv[1] - rt1)
                   + sl1(regv[2] - rt2) + sl1(regv[3] - rt3))
        obj_pred = regv[4]
        obj_bce = (_softplus(obj_pred) - obj_pred * posf) * rowmask

        npa_ref[...] += posf
        rga_ref[...] += reg_row * posf
        oba_ref[...] += obj_bce

        has_pos = jnp.max(best_iou) > 0.5
        comb_t = jnp.transpose(mcls[0:ROWS]
                               + 128.0 * posf[0:ROWS])

        if nb < NB0:
            cref, base = cls0_ref, nb * NB
        elif nb < NB0 + NB1:
            cref, base = cls1_ref, (nb - NB0) * NB
        else:
            cref, base = cls2_ref, (nb - NB0 - NB1) * NB

        @pl.when(has_pos)
        def _cls(cref=cref, base=base, comb_t=comb_t):
            acc = jnp.zeros((128, NUM_CLASSES), jnp.float32)
            for r in range(ROWS):
                cc = comb_t[:, r:r + 1]
                pf = (cc >= 128.0).astype(jnp.float32)
                cid_i = (cc - 128.0 * pf + 0.5).astype(jnp.int32)
                x = cref[0, base + r * 128: base + (r + 1) * 128, :]
                t = _softplus(x) - jnp.where(citer == cid_i, x, 0.0)
                acc = acc + t * pf
            cla_ref[...] += acc

    @pl.when(b == B - 1)
    def _fin():
        npos_ref[...] = jnp.sum(npa_ref[...]).reshape(1, 1)
        obj_ref[...] = jnp.sum(oba_ref[...]).reshape(1, 1)
        regs_ref[...] = jnp.sum(rga_ref[...]).reshape(1, 1)
        clss_ref[...] = jnp.sum(cla_ref[...]).reshape(1, 1)


@jax.jit
def _loss_pallas(tgt_s, reg_pad, cls0, cls1, cls2):
    anchors = jnp.asarray(_ANCHORS)
    grid = (B,)

    out = pl.pallas_call(
        _loss_body,
        grid=grid,
        in_specs=[
            pl.BlockSpec((1, 1, 6 * M), lambda b: (b, 0, 0),
                         memory_space=pltpu.SMEM),
            pl.BlockSpec((NB_TOT, 8, 8, 128), lambda b: (0, 0, 0, 0)),
            pl.BlockSpec((1, NB_TOT, 5, 8, 128), lambda b: (b, 0, 0, 0, 0)),
            pl.BlockSpec((1, L0, NUM_CLASSES), lambda b: (b, 0, 0)),
            pl.BlockSpec((1, L1, NUM_CLASSES), lambda b: (b, 0, 0)),
            pl.BlockSpec((1, L2, NUM_CLASSES), lambda b: (b, 0, 0)),
        ],
        out_specs=[pl.BlockSpec((1, 1), lambda b: (0, 0))] * 4,
        out_shape=[jax.ShapeDtypeStruct((1, 1), jnp.float32)] * 4,
        scratch_shapes=[
            pltpu.VMEM((8, 128), jnp.float32),
            pltpu.VMEM((8, 128), jnp.float32),
            pltpu.VMEM((8, 128), jnp.float32),
            pltpu.VMEM((128, NUM_CLASSES), jnp.float32),
        ],
        compiler_params=pltpu.CompilerParams(
            dimension_semantics=("arbitrary",)),
        interpret=_INTERPRET,
    )(tgt_s, anchors, reg_pad, cls0, cls1, cls2)
    return out


def kernel(imgs, reg_l0, reg_l1, reg_l2, cls_l0, cls_l1, cls_l2, targets):
    del imgs

    def regt(x, nblk):
        r = jnp.transpose(x.reshape(B, nblk, ROWS, 128, 5), (0, 1, 4, 2, 3))
        return jnp.concatenate(
            [r, jnp.zeros((B, nblk, 5, 8 - ROWS, 128), jnp.float32)], axis=3)

    reg_pad = jnp.concatenate(
        [regt(reg_l0, NB0), regt(reg_l1, NB1), regt(reg_l2, NB2)], axis=1)

    cls0 = cls_l0.reshape(B, L0, NUM_CLASSES)
    cls1 = cls_l1.reshape(B, L1, NUM_CLASSES)
    cls2 = cls_l2.reshape(B, L2, NUM_CLASSES)

    gx1 = targets[..., 0]
    gy1 = targets[..., 1]
    gx2 = targets[..., 2]
    gy2 = targets[..., 3]
    gcl = targets[..., 4]
    valid = jnp.logical_and(gx2 > gx1, gy2 > gy1)
    area_b = jnp.clip(gx2 - gx1, 0.0) * jnp.clip(gy2 - gy1, 0.0)
    area_b = jnp.where(valid, area_b, 1e30)
    tgt_s = jnp.stack(
        [gx1, gy1, gx2, gy2, area_b, gcl], axis=1).reshape(B, 1, 6 * M)

    npos_s, obj_s, cls_s, reg_s = _loss_pallas(
        tgt_s, reg_pad, cls0, cls1, cls2)

    npos = jnp.maximum(npos_s[0, 0], 1.0)
    loss_obj = obj_s[0, 0] / (B * N)
    loss_cls = cls_s[0, 0] / npos
    loss_reg = reg_s[0, 0] / npos
    losses = loss_reg + loss_obj + loss_cls
    return (losses, loss_reg, loss_obj, loss_cls)


# speedup vs baseline: 1.2603x; 1.0345x over previous
import numpy as np
import jax
import jax.numpy as jnp
from jax.experimental import pallas as pl
from jax.experimental.pallas import tpu as pltpu

IMG_SIZE = 512
STRIDES = [8, 16, 32]
ANCHOR_SIZES = [
    [(10.0, 13.0), (16.0, 30.0), (33.0, 23.0)],
    [(30.0, 61.0), (62.0, 45.0), (59.0, 119.0)],
    [(116.0, 90.0), (156.0, 198.0), (373.0, 326.0)],
]
NUM_CLASSES = 80
B = 8
M = 32

_INTERPRET = False

NB = 768
ROWS = NB // 128
L0 = 3 * 64 * 64
L1 = 3 * 32 * 32
L2 = 3 * 16 * 16
N = L0 + L1 + L2
NB0 = L0 // NB
NB1 = L1 // NB
NB2 = L2 // NB
NB_TOT = NB0 + NB1 + NB2


def _make_anchor_table() -> np.ndarray:
    comps = [[] for _ in range(8)]
    for stride, sizes in zip(STRIDES, ANCHOR_SIZES):
        g = IMG_SIZE // stride
        ys, xs = np.meshgrid(np.arange(g, dtype=np.float32),
                             np.arange(g, dtype=np.float32), indexing='ij')
        cx = (xs + 0.5) * stride
        cy = (ys + 0.5) * stride
        for (aw, ah) in sizes:
            x1 = (cx - aw / 2).reshape(-1)
            y1 = (cy - ah / 2).reshape(-1)
            x2 = (cx + aw / 2).reshape(-1)
            y2 = (cy + ah / 2).reshape(-1)
            vals = [x1, y1, x2, y2, (x1 + x2) / 2, (y1 + y2) / 2,
                    np.full_like(x1, aw), np.full_like(x1, ah)]
            for i in range(8):
                comps[i].append(vals[i])
    flat = np.stack([np.concatenate(c) for c in comps], axis=0)
    blocked = flat.reshape(8, NB_TOT, ROWS, 128)
    pad_row = np.zeros((8, NB_TOT, 8 - ROWS, 128), np.float32)
    pad_row[6:8] = 1.0
    out = np.concatenate([blocked, pad_row], axis=2)
    return np.ascontiguousarray(out.transpose(1, 0, 2, 3)).astype(np.float32)


_ANCHORS = _make_anchor_table()


def _softplus(x):
    return jnp.maximum(x, 0.0) + jnp.log(1.0 + jnp.exp(-jnp.abs(x)))


def _loss_body(tgt_ref, anch_ref, reg_ref, cls0_ref, cls1_ref, cls2_ref,
               npos_ref, obj_ref, clss_ref, regs_ref,
               npa_ref, oba_ref, rga_ref, cla_ref):
    b = pl.program_id(0)

    @pl.when(b == 0)
    def _init():
        npa_ref[...] = jnp.zeros_like(npa_ref)
        oba_ref[...] = jnp.zeros_like(oba_ref)
        rga_ref[...] = jnp.zeros_like(rga_ref)
        cla_ref[...] = jnp.zeros_like(cla_ref)

    rowmask = (jax.lax.broadcasted_iota(jnp.int32, (8, 128), 0)
               < ROWS).astype(jnp.float32)
    citer = jax.lax.broadcasted_iota(jnp.int32, (128, NUM_CLASSES), 1)

    for nb in range(NB_TOT):

        ax1 = anch_ref[nb, 0]
        ay1 = anch_ref[nb, 1]
        ax2 = anch_ref[nb, 2]
        ay2 = anch_ref[nb, 3]
        acx = anch_ref[nb, 4]
        acy = anch_ref[nb, 5]
        aw = anch_ref[nb, 6]
        ah = anch_ref[nb, 7]
        area_a = (ax2 - ax1) * (ay2 - ay1)

        best_iou = jnp.full((8, 128), -1.0, dtype=jnp.float32)
        mgcx = jnp.zeros((8, 128), dtype=jnp.float32)
        mgcy = jnp.zeros((8, 128), dtype=jnp.float32)
        mgw = jnp.full((8, 128), 1e-3, dtype=jnp.float32)
        mgh = jnp.full((8, 128), 1e-3, dtype=jnp.float32)
        mcls = jnp.zeros((8, 128), dtype=jnp.float32)

        for m in range(M):
            gx1 = tgt_ref[0, 0 * M + m: 0 * M + m + 1, :]
            gy1 = tgt_ref[0, 1 * M + m: 1 * M + m + 1, :]
            gx2 = tgt_ref[0, 2 * M + m: 2 * M + m + 1, :]
            gy2 = tgt_ref[0, 3 * M + m: 3 * M + m + 1, :]
            area_b = tgt_ref[0, 4 * M + m: 4 * M + m + 1, :]
            gcl = tgt_ref[0, 5 * M + m: 5 * M + m + 1, :]

            iw = jnp.clip(jnp.minimum(ax2, gx2) - jnp.maximum(ax1, gx1), 0.0)
            ih = jnp.clip(jnp.minimum(ay2, gy2) - jnp.maximum(ay1, gy1), 0.0)
            inter = iw * ih
            iou = inter / (area_a + area_b - inter + 1e-9)

            better = iou > best_iou
            best_iou = jnp.where(better, iou, best_iou)
            mgcx = jnp.where(better, (gx1 + gx2) * 0.5, mgcx)
            mgcy = jnp.where(better, (gy1 + gy2) * 0.5, mgcy)
            mgw = jnp.where(better, jnp.maximum(gx2 - gx1, 1e-3), mgw)
            mgh = jnp.where(better, jnp.maximum(gy2 - gy1, 1e-3), mgh)
            mcls = jnp.where(better, gcl, mcls)

        posf = (best_iou > 0.5).astype(jnp.float32)

        regv = reg_ref[0, nb]

        rt0 = (mgcx - acx) / aw
        rt1 = (mgcy - acy) / ah
        rt2 = jnp.log(mgw / aw)
        rt3 = jnp.log(mgh / ah)

        def sl1(d):
            ad = jnp.abs(d)
            return jnp.where(ad < 1.0, 0.5 * d * d, ad - 0.5)

        reg_row = (sl1(regv[0] - rt0) + sl1(regv[1] - rt1)
                   + sl1(regv[2] - rt2) + sl1(regv[3] - rt3))
        obj_pred = regv[4]
        obj_bce = (_softplus(obj_pred) - obj_pred * posf) * rowmask

        npa_ref[...] += posf
        rga_ref[...] += reg_row * posf
        oba_ref[...] += obj_bce

        has_pos = jnp.max(best_iou) > 0.5
        comb_t = jnp.transpose(mcls[0:ROWS]
                               + 128.0 * posf[0:ROWS])

        if nb < NB0:
            cref, base = cls0_ref, nb * NB
        elif nb < NB0 + NB1:
            cref, base = cls1_ref, (nb - NB0) * NB
        else:
            cref, base = cls2_ref, (nb - NB0 - NB1) * NB

        @pl.when(has_pos)
        def _cls(cref=cref, base=base, comb_t=comb_t):
            acc = jnp.zeros((128, NUM_CLASSES), jnp.float32)
            for r in range(ROWS):
                cc = comb_t[:, r:r + 1]
                pf = (cc >= 128.0).astype(jnp.float32)
                cid_i = (cc - 128.0 * pf + 0.5).astype(jnp.int32)
                x = cref[0, base + r * 128: base + (r + 1) * 128, :]
                t = _softplus(x) - jnp.where(citer == cid_i, x, 0.0)
                acc = acc + t * pf
            cla_ref[...] += acc

    @pl.when(b == B - 1)
    def _fin():
        npos_ref[...] = jnp.sum(npa_ref[...]).reshape(1, 1)
        obj_ref[...] = jnp.sum(oba_ref[...]).reshape(1, 1)
        regs_ref[...] = jnp.sum(rga_ref[...]).reshape(1, 1)
        clss_ref[...] = jnp.sum(cla_ref[...]).reshape(1, 1)


@jax.jit
def _loss_pallas(tgt_s, reg_pad, cls0, cls1, cls2):
    anchors = jnp.asarray(_ANCHORS)
    grid = (B,)

    out = pl.pallas_call(
        _loss_body,
        grid=grid,
        in_specs=[
            pl.BlockSpec((1, 6 * M, 128), lambda b: (b, 0, 0)),
            pl.BlockSpec((NB_TOT, 8, 8, 128), lambda b: (0, 0, 0, 0)),
            pl.BlockSpec((1, NB_TOT, 5, 8, 128), lambda b: (b, 0, 0, 0, 0)),
            pl.BlockSpec((1, L0, NUM_CLASSES), lambda b: (b, 0, 0)),
            pl.BlockSpec((1, L1, NUM_CLASSES), lambda b: (b, 0, 0)),
            pl.BlockSpec((1, L2, NUM_CLASSES), lambda b: (b, 0, 0)),
        ],
        out_specs=[pl.BlockSpec((1, 1), lambda b: (0, 0))] * 4,
        out_shape=[jax.ShapeDtypeStruct((1, 1), jnp.float32)] * 4,
        scratch_shapes=[
            pltpu.VMEM((8, 128), jnp.float32),
            pltpu.VMEM((8, 128), jnp.float32),
            pltpu.VMEM((8, 128), jnp.float32),
            pltpu.VMEM((128, NUM_CLASSES), jnp.float32),
        ],
        compiler_params=pltpu.CompilerParams(
            dimension_semantics=("arbitrary",)),
        interpret=_INTERPRET,
    )(tgt_s, anchors, reg_pad, cls0, cls1, cls2)
    return out


def kernel(imgs, reg_l0, reg_l1, reg_l2, cls_l0, cls_l1, cls_l2, targets):
    del imgs

    def regt(x, nblk):
        r = jnp.transpose(x.reshape(B, nblk, ROWS, 128, 5), (0, 1, 4, 2, 3))
        return jnp.concatenate(
            [r, jnp.zeros((B, nblk, 5, 8 - ROWS, 128), jnp.float32)], axis=3)

    reg_pad = jnp.concatenate(
        [regt(reg_l0, NB0), regt(reg_l1, NB1), regt(reg_l2, NB2)], axis=1)

    cls0 = cls_l0.reshape(B, L0, NUM_CLASSES)
    cls1 = cls_l1.reshape(B, L1, NUM_CLASSES)
    cls2 = cls_l2.reshape(B, L2, NUM_CLASSES)

    gx1 = targets[..., 0]
    gy1 = targets[..., 1]
    gx2 = targets[..., 2]
    gy2 = targets[..., 3]
    gcl = targets[..., 4]
    valid = jnp.logical_and(gx2 > gx1, gy2 > gy1)
    area_b = jnp.clip(gx2 - gx1, 0.0) * jnp.clip(gy2 - gy1, 0.0)
    area_b = jnp.where(valid, area_b, 1e30)
    tgt_s = jnp.broadcast_to(
        jnp.stack([gx1, gy1, gx2, gy2, area_b, gcl],
                  axis=1).reshape(B, 6 * M, 1), (B, 6 * M, 128))

    npos_s, obj_s, cls_s, reg_s = _loss_pallas(
        tgt_s, reg_pad, cls0, cls1, cls2)

    npos = jnp.maximum(npos_s[0, 0], 1.0)
    loss_obj = obj_s[0, 0] / (B * N)
    loss_cls = cls_s[0, 0] / npos
    loss_reg = reg_s[0, 0] / npos
    losses = loss_reg + loss_obj + loss_cls
    return (losses, loss_reg, loss_obj, loss_cls)
